# baseline (speedup 1.0000x reference)
import jax
import jax.numpy as jnp
from jax.experimental import pallas as pl
from jax.experimental.pallas import tpu as pltpu

MAX_LEN = 50
BATCH = 1024
N_RULES = 256
N_TOKENS = 1000
COPY_LEN = 200

_CHUNKS = 8
_BS = BATCH // _CHUNKS


def _copy_body(s_ref, r_in, t_in, c_in, r_out, t_out, c_out):
    del s_ref
    r_out[...] = r_in[0]
    t_out[...] = t_in[0]
    c_out[...] = c_in[0]


def kernel(rule_prob, token_prob, copy_prob, length):
    def im_in(i, s):
        idx = (s[0] - 1) % MAX_LEN
        return (idx, i, 0)

    def im_out(i, s):
        del s
        return (i, 0)

    grid_spec = pltpu.PrefetchScalarGridSpec(
        num_scalar_prefetch=1,
        grid=(_CHUNKS,),
        in_specs=[
            pl.BlockSpec((1, _BS, N_RULES), im_in),
            pl.BlockSpec((1, _BS, N_TOKENS), im_in),
            pl.BlockSpec((1, _BS, COPY_LEN), im_in),
        ],
        out_specs=[
            pl.BlockSpec((_BS, N_RULES), im_out),
            pl.BlockSpec((_BS, N_TOKENS), im_out),
            pl.BlockSpec((_BS, COPY_LEN), im_out),
        ],
    )
    out_shape = [
        jax.ShapeDtypeStruct((BATCH, N_RULES), jnp.float32),
        jax.ShapeDtypeStruct((BATCH, N_TOKENS), jnp.float32),
        jax.ShapeDtypeStruct((BATCH, COPY_LEN), jnp.float32),
    ]
    r, t, c = pl.pallas_call(
        _copy_body, grid_spec=grid_spec, out_shape=out_shape
    )(length, rule_prob, token_prob, copy_prob)
    return (r, t, c)
